# baseline (speedup 1.0000x reference)
import functools

import jax
import jax.numpy as jnp
from jax import lax
from jax.experimental import pallas as pl
from jax.experimental.pallas import tpu as pltpu
from jax.experimental.pallas import tpu_sc as plsc

N = 100000
NP = 100096
E = 3200000
EPAD = 3276800
ER = EPAD // 128
RPT = NP // 16
DRB = 391
BB = 5
BN = 4352
GRID = NP // BN

_mesh = plsc.VectorSubcoreMesh(core_axis_name="c", subcore_axis_name="s")
_sc_params = pltpu.CompilerParams(use_tc_tiling_on_sc=False)


@functools.partial(
    pl.kernel,
    out_type=jax.ShapeDtypeStruct((2, NP), jnp.float32),
    mesh=_mesh,
    compiler_params=_sc_params,
    scratch_types=[
        pltpu.VMEM((128,), jnp.float32),
        pltpu.VMEM((RPT,), jnp.float32),
        pltpu.VMEM((16, 128), jnp.int32),
        pltpu.VMEM_SHARED((NP,), jnp.float32),
    ],
)
def _deg_sc(dst_hbm, ones_hbm, zeros_hbm, out_hbm, ones_v, zbuf, dstv, acc):
    c = lax.axis_index("c")
    s = lax.axis_index("s")
    pltpu.sync_copy(ones_hbm, ones_v)
    pltpu.sync_copy(zeros_hbm, zbuf)
    pltpu.sync_copy(zbuf, acc.at[pl.ds(s * RPT, RPT)])
    plsc.subcore_barrier()
    base = (s * 2 + c) * 800

    def chunk(i, _):
        off = base + i * 16
        pltpu.sync_copy(dst_hbm.at[pl.ds(off, 16), :], dstv)
        for j in range(16):
            pltpu.sync_copy(ones_v, acc.at[dstv.at[j]], add=True)
        return 0

    lax.fori_loop(0, 50, chunk, 0)
    plsc.subcore_barrier()
    pltpu.sync_copy(acc.at[pl.ds(s * RPT, RPT)], zbuf)
    pltpu.sync_copy(zbuf, out_hbm.at[c, pl.ds(s * RPT, RPT)])


def _make_row_agg(n_tables, c_stride, s_stride, n_chunks):
    @functools.partial(
        pl.kernel,
        out_type=jax.ShapeDtypeStruct((2, NP, 16), jnp.float32),
        mesh=_mesh,
        compiler_params=_sc_params,
        scratch_types=[
            pltpu.VMEM((DRB, 16), jnp.float32),
            pltpu.VMEM((2, BB, 128), jnp.int32),
            pltpu.VMEM((2, BB, 128), jnp.int32),
            pltpu.VMEM((2, BB, 128, 16), jnp.float32),
            pltpu.SemaphoreType.DMA((2, BB)),
            pltpu.SemaphoreType.DMA((2, BB)),
            pltpu.VMEM_SHARED((NP, 16), jnp.float32),
        ],
    )
    def agg(table_hbm, src_hbm, dst_hbm, zeros_hbm, out_hbm,
            zbuf, srcv, dstv, rows, gsems, ssems, acc):
        c = lax.axis_index("c")
        s = lax.axis_index("s")
        table = table_hbm.at[c * (n_tables - 1)]
        pltpu.sync_copy(zeros_hbm, zbuf)
        for k in range(RPT // DRB):
            pltpu.sync_copy(zbuf, acc.at[pl.ds(s * RPT + k * DRB, DRB), :])
        plsc.subcore_barrier()
        base = c * c_stride + s * s_stride

        def chunk(i, _):
            ph = lax.rem(i, 2)
            off = base + i * BB

            @pl.when(i >= 2)
            def _():
                for j in range(BB):
                    pltpu.make_async_copy(
                        rows.at[ph, j], acc.at[dstv.at[ph, j]],
                        ssems.at[ph, j]).wait()

            pltpu.sync_copy(src_hbm.at[pl.ds(off, BB), :], srcv.at[ph])
            pltpu.sync_copy(dst_hbm.at[pl.ds(off, BB), :], dstv.at[ph])
            gd = [
                pltpu.async_copy(
                    table.at[srcv.at[ph, j]], rows.at[ph, j], gsems.at[ph, j])
                for j in range(BB)
            ]
            for j in range(BB):
                gd[j].wait()
                pltpu.async_copy(
                    rows.at[ph, j], acc.at[dstv.at[ph, j]], ssems.at[ph, j],
                    add=True)
            return 0

        lax.fori_loop(0, n_chunks, chunk, 0)
        for ph in range(2):
            for j in range(BB):
                pltpu.make_async_copy(
                    rows.at[ph, j], acc.at[dstv.at[ph, j]], ssems.at[ph, j]
                ).wait()
        plsc.subcore_barrier()
        for k in range(RPT // DRB):
            r0 = s * RPT + k * DRB
            pltpu.sync_copy(acc.at[pl.ds(r0, DRB), :], zbuf)
            pltpu.sync_copy(zbuf, out_hbm.at[c, pl.ds(r0, DRB), :])

    return agg


_agg_l1 = _make_row_agg(1, 800, 1600, 160)
_agg_l2 = _make_row_agg(2, 0, 1600, 320)


def _t0_body(deg_ref, x_ref, g1_ref):
    dis = lax.rsqrt(deg_ref[0] + deg_ref[1] + 1.0)
    g1_ref[...] = x_ref[...] * dis[:, None]


def _t0(deg, x_p):
    return pl.pallas_call(
        _t0_body,
        grid=(GRID,),
        in_specs=[
            pl.BlockSpec((2, BN), lambda i: (0, i)),
            pl.BlockSpec((BN, 16), lambda i: (i, 0)),
        ],
        out_specs=pl.BlockSpec((BN, 16), lambda i: (i, 0)),
        out_shape=jax.ShapeDtypeStruct((NP, 16), jnp.float32),
    )(deg, x_p)


def _t1_body(deg_ref, s1_ref, g1_ref, w1_ref, b1_ref, w2_ref, g2_ref):
    dis = lax.rsqrt(deg_ref[0] + deg_ref[1] + 1.0)[:, None]
    y1 = (dis * (s1_ref[0] + s1_ref[1] + g1_ref[...])) @ w1_ref[...] + b1_ref[...]
    t = jax.nn.relu(y1) @ w2_ref[...]
    g2 = t * dis
    g2_ref[0] = g2[:, :16]
    g2_ref[1] = g2[:, 16:]


def _t1(deg, s1, g1, W1, b1, W2):
    return pl.pallas_call(
        _t1_body,
        grid=(GRID,),
        in_specs=[
            pl.BlockSpec((2, BN), lambda i: (0, i)),
            pl.BlockSpec((2, BN, 16), lambda i: (0, i, 0)),
            pl.BlockSpec((BN, 16), lambda i: (i, 0)),
            pl.BlockSpec((16, 48), lambda i: (0, 0)),
            pl.BlockSpec((48,), lambda i: (0,)),
            pl.BlockSpec((48, 32), lambda i: (0, 0)),
        ],
        out_specs=pl.BlockSpec((2, BN, 16), lambda i: (0, i, 0)),
        out_shape=jax.ShapeDtypeStruct((2, NP, 16), jnp.float32),
    )(deg, s1, g1, W1, b1, W2)


def _t2_body(deg_ref, s2_ref, g2_ref, b2_ref, w3_ref, b3_ref, o_ref):
    dis = lax.rsqrt(deg_ref[0] + deg_ref[1] + 1.0)[:, None]
    ya = dis * (s2_ref[0] + g2_ref[0])
    yb = dis * (s2_ref[1] + g2_ref[1])
    h2 = jax.nn.relu(jnp.concatenate([ya, yb], axis=1) + b2_ref[...])
    o_ref[...] = h2 @ w3_ref[...] + b3_ref[...]


def _t2(deg, s2, g2, b2, W3, b3):
    return pl.pallas_call(
        _t2_body,
        grid=(GRID,),
        in_specs=[
            pl.BlockSpec((2, BN), lambda i: (0, i)),
            pl.BlockSpec((2, BN, 16), lambda i: (0, i, 0)),
            pl.BlockSpec((2, BN, 16), lambda i: (0, i, 0)),
            pl.BlockSpec((32,), lambda i: (0,)),
            pl.BlockSpec((32, 10), lambda i: (0, 0)),
            pl.BlockSpec((10,), lambda i: (0,)),
        ],
        out_specs=pl.BlockSpec((BN, 10), lambda i: (i, 0)),
        out_shape=jax.ShapeDtypeStruct((NP, 10), jnp.float32),
    )(deg, s2, g2, b2, W3, b3)


def kernel(x, edge_index, W1, b1, W2, b2, W3, b3):
    x = x.astype(jnp.float32)
    ei = edge_index.astype(jnp.int32)
    pad = jnp.full((EPAD - E,), N, jnp.int32)
    srcr = jnp.concatenate([ei[0], pad]).reshape(ER, 128)
    dstr = jnp.concatenate([ei[1], pad]).reshape(ER, 128)
    x_p = jnp.zeros((NP, 16), jnp.float32).at[:N].set(x)

    ones128 = jnp.ones((128,), jnp.float32)
    zeros1 = jnp.zeros((RPT,), jnp.float32)
    zeros16 = jnp.zeros((DRB, 16), jnp.float32)

    deg = _deg_sc(dstr, ones128, zeros1)
    g1 = _t0(deg, x_p)
    s1 = _agg_l1(g1.reshape(1, NP, 16), srcr, dstr, zeros16)
    g2 = _t1(deg, s1, g1, W1, b1, W2)
    s2 = _agg_l2(g2, srcr, dstr, zeros16)
    out = _t2(deg, s2, g2, b2, W3, b3)
    return out[:N]


# speedup vs baseline: 1.0492x; 1.0492x over previous
import functools

import jax
import jax.numpy as jnp
from jax import lax
from jax.experimental import pallas as pl
from jax.experimental.pallas import tpu as pltpu
from jax.experimental.pallas import tpu_sc as plsc

N = 100000
NP = 100096
E = 3200000
EPAD = 3276800
ER = EPAD // 128
RPT = NP // 16
DRB = 391
BB = 8
BN = 4352
GRID = NP // BN

_mesh = plsc.VectorSubcoreMesh(core_axis_name="c", subcore_axis_name="s")
_sc_params = pltpu.CompilerParams(use_tc_tiling_on_sc=False)


@functools.partial(
    pl.kernel,
    out_type=jax.ShapeDtypeStruct((2, NP), jnp.float32),
    mesh=_mesh,
    compiler_params=_sc_params,
    scratch_types=[
        pltpu.VMEM((128,), jnp.float32),
        pltpu.VMEM((RPT,), jnp.float32),
        pltpu.VMEM((16, 128), jnp.int32),
        pltpu.VMEM_SHARED((NP,), jnp.float32),
    ],
)
def _deg_sc(dst_hbm, ones_hbm, zeros_hbm, out_hbm, ones_v, zbuf, dstv, acc):
    c = lax.axis_index("c")
    s = lax.axis_index("s")
    pltpu.sync_copy(ones_hbm, ones_v)
    pltpu.sync_copy(zeros_hbm, zbuf)
    pltpu.sync_copy(zbuf, acc.at[pl.ds(s * RPT, RPT)])
    plsc.subcore_barrier()
    base = (s * 2 + c) * 800

    def chunk(i, _):
        off = base + i * 16
        pltpu.sync_copy(dst_hbm.at[pl.ds(off, 16), :], dstv)
        for j in range(16):
            pltpu.sync_copy(ones_v, acc.at[dstv.at[j]], add=True)
        return 0

    lax.fori_loop(0, 50, chunk, 0)
    plsc.subcore_barrier()
    pltpu.sync_copy(acc.at[pl.ds(s * RPT, RPT)], zbuf)
    pltpu.sync_copy(zbuf, out_hbm.at[c, pl.ds(s * RPT, RPT)])


def _make_row_agg(n_tables, c_stride, s_stride, n_chunks):
    @functools.partial(
        pl.kernel,
        out_type=jax.ShapeDtypeStruct((2, NP, 16), jnp.float32),
        mesh=_mesh,
        compiler_params=_sc_params,
        scratch_types=[
            pltpu.VMEM((DRB, 16), jnp.float32),
            pltpu.VMEM((BB, 128), jnp.int32),
            pltpu.VMEM((BB, 128), jnp.int32),
            pltpu.VMEM((BB, 128, 16), jnp.float32),
            pltpu.SemaphoreType.DMA((BB,)),
            pltpu.SemaphoreType.DMA((BB,)),
            pltpu.VMEM_SHARED((NP, 16), jnp.float32),
        ],
    )
    def agg(table_hbm, src_hbm, dst_hbm, zeros_hbm, out_hbm,
            zbuf, srcv, dstv, rows, gsems, ssems, acc):
        c = lax.axis_index("c")
        s = lax.axis_index("s")
        table = table_hbm.at[c * (n_tables - 1)]
        pltpu.sync_copy(zeros_hbm, zbuf)
        for k in range(RPT // DRB):
            pltpu.sync_copy(zbuf, acc.at[pl.ds(s * RPT + k * DRB, DRB), :])
        plsc.subcore_barrier()
        base = c * c_stride + s * s_stride

        def chunk(i, _):
            off = base + i * BB
            pltpu.sync_copy(src_hbm.at[pl.ds(off, BB), :], srcv)
            pltpu.sync_copy(dst_hbm.at[pl.ds(off, BB), :], dstv)
            gd = [
                pltpu.async_copy(table.at[srcv.at[j]], rows.at[j], gsems.at[j])
                for j in range(BB)
            ]
            sd = []
            for j in range(BB):
                gd[j].wait()
                sd.append(pltpu.async_copy(
                    rows.at[j], acc.at[dstv.at[j]], ssems.at[j], add=True))
            for j in range(BB):
                sd[j].wait()
            return 0

        lax.fori_loop(0, n_chunks, chunk, 0)
        plsc.subcore_barrier()
        for k in range(RPT // DRB):
            r0 = s * RPT + k * DRB
            pltpu.sync_copy(acc.at[pl.ds(r0, DRB), :], zbuf)
            pltpu.sync_copy(zbuf, out_hbm.at[c, pl.ds(r0, DRB), :])

    return agg


_agg_l1 = _make_row_agg(1, 800, 1600, 100)
_agg_l2 = _make_row_agg(2, 0, 1600, 200)


def _t0_body(deg_ref, x_ref, g1_ref):
    dis = lax.rsqrt(deg_ref[0] + deg_ref[1] + 1.0)
    g1_ref[...] = x_ref[...] * dis[:, None]


def _t0(deg, x_p):
    return pl.pallas_call(
        _t0_body,
        grid=(GRID,),
        in_specs=[
            pl.BlockSpec((2, BN), lambda i: (0, i)),
            pl.BlockSpec((BN, 16), lambda i: (i, 0)),
        ],
        out_specs=pl.BlockSpec((BN, 16), lambda i: (i, 0)),
        out_shape=jax.ShapeDtypeStruct((NP, 16), jnp.float32),
    )(deg, x_p)


def _t1_body(deg_ref, s1_ref, g1_ref, w1_ref, b1_ref, w2_ref, g2_ref):
    dis = lax.rsqrt(deg_ref[0] + deg_ref[1] + 1.0)[:, None]
    y1 = (dis * (s1_ref[0] + s1_ref[1] + g1_ref[...])) @ w1_ref[...] + b1_ref[...]
    t = jax.nn.relu(y1) @ w2_ref[...]
    g2 = t * dis
    g2_ref[0] = g2[:, :16]
    g2_ref[1] = g2[:, 16:]


def _t1(deg, s1, g1, W1, b1, W2):
    return pl.pallas_call(
        _t1_body,
        grid=(GRID,),
        in_specs=[
            pl.BlockSpec((2, BN), lambda i: (0, i)),
            pl.BlockSpec((2, BN, 16), lambda i: (0, i, 0)),
            pl.BlockSpec((BN, 16), lambda i: (i, 0)),
            pl.BlockSpec((16, 48), lambda i: (0, 0)),
            pl.BlockSpec((48,), lambda i: (0,)),
            pl.BlockSpec((48, 32), lambda i: (0, 0)),
        ],
        out_specs=pl.BlockSpec((2, BN, 16), lambda i: (0, i, 0)),
        out_shape=jax.ShapeDtypeStruct((2, NP, 16), jnp.float32),
    )(deg, s1, g1, W1, b1, W2)


def _t2_body(deg_ref, s2_ref, g2_ref, b2_ref, w3_ref, b3_ref, o_ref):
    dis = lax.rsqrt(deg_ref[0] + deg_ref[1] + 1.0)[:, None]
    ya = dis * (s2_ref[0] + g2_ref[0])
    yb = dis * (s2_ref[1] + g2_ref[1])
    h2 = jax.nn.relu(jnp.concatenate([ya, yb], axis=1) + b2_ref[...])
    o_ref[...] = h2 @ w3_ref[...] + b3_ref[...]


def _t2(deg, s2, g2, b2, W3, b3):
    return pl.pallas_call(
        _t2_body,
        grid=(GRID,),
        in_specs=[
            pl.BlockSpec((2, BN), lambda i: (0, i)),
            pl.BlockSpec((2, BN, 16), lambda i: (0, i, 0)),
            pl.BlockSpec((2, BN, 16), lambda i: (0, i, 0)),
            pl.BlockSpec((32,), lambda i: (0,)),
            pl.BlockSpec((32, 10), lambda i: (0, 0)),
            pl.BlockSpec((10,), lambda i: (0,)),
        ],
        out_specs=pl.BlockSpec((BN, 10), lambda i: (i, 0)),
        out_shape=jax.ShapeDtypeStruct((NP, 10), jnp.float32),
    )(deg, s2, g2, b2, W3, b3)


def kernel(x, edge_index, W1, b1, W2, b2, W3, b3):
    x = x.astype(jnp.float32)
    ei = edge_index.astype(jnp.int32)
    pad = jnp.full((EPAD - E,), N, jnp.int32)
    srcr = jnp.concatenate([ei[0], pad]).reshape(ER, 128)
    dstr = jnp.concatenate([ei[1], pad]).reshape(ER, 128)
    x_p = jnp.zeros((NP, 16), jnp.float32).at[:N].set(x)

    ones128 = jnp.ones((128,), jnp.float32)
    zeros1 = jnp.zeros((RPT,), jnp.float32)
    zeros16 = jnp.zeros((DRB, 16), jnp.float32)

    deg = _deg_sc(dstr, ones128, zeros1)
    g1 = _t0(deg, x_p)
    s1 = _agg_l1(g1.reshape(1, NP, 16), srcr, dstr, zeros16)
    g2 = _t1(deg, s1, g1, W1, b1, W2)
    s2 = _agg_l2(g2, srcr, dstr, zeros16)
    out = _t2(deg, s2, g2, b2, W3, b3)
    return out[:N]


# speedup vs baseline: 1.2389x; 1.1808x over previous
import functools

import jax
import jax.numpy as jnp
from jax import lax
from jax.experimental import pallas as pl
from jax.experimental.pallas import tpu as pltpu
from jax.experimental.pallas import tpu_sc as plsc

N = 100000
NP = 100096
E = 3200000
EPAD = 3276800
ER = EPAD // 128
RPT = NP // 16
DRB = 391
BB = 5
BN = 4352
GRID = NP // BN

_mesh = plsc.VectorSubcoreMesh(core_axis_name="c", subcore_axis_name="s")
_sc_params = pltpu.CompilerParams(use_tc_tiling_on_sc=False)


@functools.partial(
    pl.kernel,
    out_type=jax.ShapeDtypeStruct((2, NP), jnp.float32),
    mesh=_mesh,
    compiler_params=_sc_params,
    scratch_types=[
        pltpu.VMEM((128,), jnp.float32),
        pltpu.VMEM((RPT,), jnp.float32),
        pltpu.VMEM((16, 128), jnp.int32),
        pltpu.SemaphoreType.DMA((16,)),
        pltpu.VMEM_SHARED((NP,), jnp.float32),
    ],
)
def _deg_sc(dst_hbm, ones_hbm, zeros_hbm, out_hbm, ones_v, zbuf, dstv, sems, acc):
    c = lax.axis_index("c")
    s = lax.axis_index("s")
    pltpu.sync_copy(ones_hbm, ones_v)
    pltpu.sync_copy(zeros_hbm, zbuf)
    pltpu.sync_copy(zbuf, acc.at[pl.ds(s * RPT, RPT)])
    plsc.subcore_barrier()
    base = (s * 2 + c) * 800

    def chunk(i, _):
        off = base + i * 16
        pltpu.sync_copy(dst_hbm.at[pl.ds(off, 16), :], dstv)
        sd = [
            pltpu.async_copy(ones_v, acc.at[dstv.at[j]], sems.at[j], add=True)
            for j in range(16)
        ]
        for j in range(16):
            sd[j].wait()
        return 0

    lax.fori_loop(0, 50, chunk, 0)
    plsc.subcore_barrier()
    pltpu.sync_copy(acc.at[pl.ds(s * RPT, RPT)], zbuf)
    pltpu.sync_copy(zbuf, out_hbm.at[c, pl.ds(s * RPT, RPT)])


def _make_row_agg(n_tables, c_stride, s_stride, n_chunks):
    @functools.partial(
        pl.kernel,
        out_type=jax.ShapeDtypeStruct((2, NP, 16), jnp.float32),
        mesh=_mesh,
        compiler_params=_sc_params,
        scratch_types=[
            pltpu.VMEM((DRB, 16), jnp.float32),
            pltpu.VMEM((2, BB, 2, 128), jnp.int32),
            pltpu.VMEM((2, BB, 128, 16), jnp.float32),
            pltpu.SemaphoreType.DMA((2, BB)),
            pltpu.SemaphoreType.DMA((2, BB)),
            pltpu.VMEM_SHARED((NP, 16), jnp.float32),
        ],
    )
    def agg(table_hbm, eidx_hbm, zeros_hbm, out_hbm,
            zbuf, idxv, rows, gsems, ssems, acc):
        c = lax.axis_index("c")
        s = lax.axis_index("s")
        table = table_hbm.at[c * (n_tables - 1)]
        pltpu.sync_copy(zeros_hbm, zbuf)
        for k in range(RPT // DRB):
            pltpu.sync_copy(zbuf, acc.at[pl.ds(s * RPT + k * DRB, DRB), :])
        plsc.subcore_barrier()
        base = c * c_stride + s * s_stride

        def load_idx(ph, off):
            pltpu.sync_copy(eidx_hbm.at[pl.ds(off, BB), :, :], idxv.at[ph])

        def issue_gathers(ph):
            for j in range(BB):
                pltpu.async_copy(
                    table.at[idxv.at[ph, j, 0]], rows.at[ph, j],
                    gsems.at[ph, j])

        def wait_gathers_issue_scatters(ph):
            for j in range(BB):
                pltpu.make_async_copy(
                    table.at[idxv.at[ph, j, 0]], rows.at[ph, j],
                    gsems.at[ph, j]).wait()
                pltpu.async_copy(
                    rows.at[ph, j], acc.at[idxv.at[ph, j, 1]],
                    ssems.at[ph, j], add=True)

        def wait_scatters(ph):
            for j in range(BB):
                pltpu.make_async_copy(
                    rows.at[ph, j], acc.at[idxv.at[ph, j, 1]],
                    ssems.at[ph, j]).wait()

        load_idx(0, base)
        issue_gathers(0)
        load_idx(1, base + BB)
        issue_gathers(1)

        def pair(p, _):
            for ph in range(2):
                wait_gathers_issue_scatters(ph)
                wait_scatters(ph)
                load_idx(ph, base + (2 * p + ph + 2) * BB)
                issue_gathers(ph)
            return 0

        lax.fori_loop(0, n_chunks // 2 - 1, pair, 0)
        for ph in range(2):
            wait_gathers_issue_scatters(ph)
            wait_scatters(ph)
        plsc.subcore_barrier()
        for k in range(RPT // DRB):
            r0 = s * RPT + k * DRB
            pltpu.sync_copy(acc.at[pl.ds(r0, DRB), :], zbuf)
            pltpu.sync_copy(zbuf, out_hbm.at[c, pl.ds(r0, DRB), :])

    return agg


_agg_l1 = _make_row_agg(1, 800, 1600, 160)
_agg_l2 = _make_row_agg(2, 0, 1600, 320)


def _t0_body(deg_ref, x_ref, g1_ref):
    dis = lax.rsqrt(deg_ref[0] + deg_ref[1] + 1.0)
    g1_ref[...] = x_ref[...] * dis[:, None]


def _t0(deg, x_p):
    return pl.pallas_call(
        _t0_body,
        grid=(GRID,),
        in_specs=[
            pl.BlockSpec((2, BN), lambda i: (0, i)),
            pl.BlockSpec((BN, 16), lambda i: (i, 0)),
        ],
        out_specs=pl.BlockSpec((BN, 16), lambda i: (i, 0)),
        out_shape=jax.ShapeDtypeStruct((NP, 16), jnp.float32),
    )(deg, x_p)


def _t1_body(deg_ref, s1_ref, g1_ref, w1_ref, b1_ref, w2_ref, g2_ref):
    dis = lax.rsqrt(deg_ref[0] + deg_ref[1] + 1.0)[:, None]
    y1 = (dis * (s1_ref[0] + s1_ref[1] + g1_ref[...])) @ w1_ref[...] + b1_ref[...]
    t = jax.nn.relu(y1) @ w2_ref[...]
    g2 = t * dis
    g2_ref[0] = g2[:, :16]
    g2_ref[1] = g2[:, 16:]


def _t1(deg, s1, g1, W1, b1, W2):
    return pl.pallas_call(
        _t1_body,
        grid=(GRID,),
        in_specs=[
            pl.BlockSpec((2, BN), lambda i: (0, i)),
            pl.BlockSpec((2, BN, 16), lambda i: (0, i, 0)),
            pl.BlockSpec((BN, 16), lambda i: (i, 0)),
            pl.BlockSpec((16, 48), lambda i: (0, 0)),
            pl.BlockSpec((48,), lambda i: (0,)),
            pl.BlockSpec((48, 32), lambda i: (0, 0)),
        ],
        out_specs=pl.BlockSpec((2, BN, 16), lambda i: (0, i, 0)),
        out_shape=jax.ShapeDtypeStruct((2, NP, 16), jnp.float32),
    )(deg, s1, g1, W1, b1, W2)


def _t2_body(deg_ref, s2_ref, g2_ref, b2_ref, w3_ref, b3_ref, o_ref):
    dis = lax.rsqrt(deg_ref[0] + deg_ref[1] + 1.0)[:, None]
    ya = dis * (s2_ref[0] + g2_ref[0])
    yb = dis * (s2_ref[1] + g2_ref[1])
    h2 = jax.nn.relu(jnp.concatenate([ya, yb], axis=1) + b2_ref[...])
    o_ref[...] = h2 @ w3_ref[...] + b3_ref[...]


def _t2(deg, s2, g2, b2, W3, b3):
    return pl.pallas_call(
        _t2_body,
        grid=(GRID,),
        in_specs=[
            pl.BlockSpec((2, BN), lambda i: (0, i)),
            pl.BlockSpec((2, BN, 16), lambda i: (0, i, 0)),
            pl.BlockSpec((2, BN, 16), lambda i: (0, i, 0)),
            pl.BlockSpec((32,), lambda i: (0,)),
            pl.BlockSpec((32, 10), lambda i: (0, 0)),
            pl.BlockSpec((10,), lambda i: (0,)),
        ],
        out_specs=pl.BlockSpec((BN, 10), lambda i: (i, 0)),
        out_shape=jax.ShapeDtypeStruct((NP, 10), jnp.float32),
    )(deg, s2, g2, b2, W3, b3)


def kernel(x, edge_index, W1, b1, W2, b2, W3, b3):
    x = x.astype(jnp.float32)
    ei = edge_index.astype(jnp.int32)
    pad = jnp.full((EPAD - E,), N, jnp.int32)
    srcr = jnp.concatenate([ei[0], pad]).reshape(ER, 128)
    dstr = jnp.concatenate([ei[1], pad]).reshape(ER, 128)
    x_p = jnp.zeros((NP, 16), jnp.float32).at[:N].set(x)

    ones128 = jnp.ones((128,), jnp.float32)
    zeros1 = jnp.zeros((RPT,), jnp.float32)
    zeros16 = jnp.zeros((DRB, 16), jnp.float32)

    eidx = jnp.stack([srcr, dstr], axis=1)
    deg = _deg_sc(dstr, ones128, zeros1)
    g1 = _t0(deg, x_p)
    s1 = _agg_l1(g1.reshape(1, NP, 16), eidx, zeros16)
    g2 = _t1(deg, s1, g1, W1, b1, W2)
    s2 = _agg_l2(g2, eidx, zeros16)
    out = _t2(deg, s2, g2, b2, W3, b3)
    return out[:N]


# speedup vs baseline: 1.2560x; 1.0138x over previous
import functools

import jax
import jax.numpy as jnp
from jax import lax
from jax.experimental import pallas as pl
from jax.experimental.pallas import tpu as pltpu
from jax.experimental.pallas import tpu_sc as plsc

N = 100000
NP = 100096
E = 3200000
EPAD = 3276800
ER = EPAD // 128
RPT = NP // 16
DRB = 391
BB = 5
BN = 4352
GRID = NP // BN

_mesh = plsc.VectorSubcoreMesh(core_axis_name="c", subcore_axis_name="s")
_sc_params = pltpu.CompilerParams(use_tc_tiling_on_sc=False)


@functools.partial(
    pl.kernel,
    out_type=jax.ShapeDtypeStruct((2, NP), jnp.float32),
    mesh=_mesh,
    compiler_params=_sc_params,
    scratch_types=[
        pltpu.VMEM((128,), jnp.float32),
        pltpu.VMEM((RPT,), jnp.float32),
        pltpu.VMEM((16, 128), jnp.int32),
        pltpu.SemaphoreType.DMA((16,)),
        pltpu.VMEM_SHARED((NP,), jnp.float32),
    ],
)
def _deg_sc(dst_hbm, ones_hbm, zeros_hbm, out_hbm, ones_v, zbuf, dstv, sems, acc):
    c = lax.axis_index("c")
    s = lax.axis_index("s")
    pltpu.sync_copy(ones_hbm, ones_v)
    pltpu.sync_copy(zeros_hbm, zbuf)
    pltpu.sync_copy(zbuf, acc.at[pl.ds(s * RPT, RPT)])
    plsc.subcore_barrier()
    base = (s * 2 + c) * 800

    def chunk(i, _):
        off = base + i * 16
        pltpu.sync_copy(dst_hbm.at[pl.ds(off, 16), :], dstv)
        sd = [
            pltpu.async_copy(ones_v, acc.at[dstv.at[j]], sems.at[j], add=True)
            for j in range(16)
        ]
        for j in range(16):
            sd[j].wait()
        return 0

    lax.fori_loop(0, 50, chunk, 0)
    plsc.subcore_barrier()
    pltpu.sync_copy(acc.at[pl.ds(s * RPT, RPT)], zbuf)
    pltpu.sync_copy(zbuf, out_hbm.at[c, pl.ds(s * RPT, RPT)])


def _make_row_agg(n_tables, c_stride, s_stride, n_chunks):
    @functools.partial(
        pl.kernel,
        out_type=jax.ShapeDtypeStruct((2, NP, 16), jnp.float32),
        mesh=_mesh,
        compiler_params=_sc_params,
        scratch_types=[
            pltpu.VMEM((DRB, 16), jnp.float32),
            pltpu.VMEM((2, BB, 2, 128), jnp.int32),
            pltpu.VMEM((2, BB, 128, 16), jnp.float32),
            pltpu.SemaphoreType.DMA((2, BB)),
            pltpu.SemaphoreType.DMA((2, BB)),
            pltpu.VMEM_SHARED((NP, 16), jnp.float32),
        ],
    )
    def agg(table_hbm, eidx_hbm, zeros_hbm, out_hbm,
            zbuf, idxv, rows, gsems, ssems, acc):
        c = lax.axis_index("c")
        s = lax.axis_index("s")
        table = table_hbm.at[c * (n_tables - 1)]
        pltpu.sync_copy(zeros_hbm, zbuf)
        for k in range(RPT // DRB):
            pltpu.sync_copy(zbuf, acc.at[pl.ds(s * RPT + k * DRB, DRB), :])
        plsc.subcore_barrier()
        base = c * c_stride + s * s_stride

        def load_idx(ph, off):
            pltpu.sync_copy(eidx_hbm.at[pl.ds(off, BB), :, :], idxv.at[ph])

        def issue_gathers(ph):
            for j in range(BB):
                pltpu.async_copy(
                    table.at[idxv.at[ph, j, 0]], rows.at[ph, j],
                    gsems.at[ph, j])

        def wait_gathers_issue_scatters(ph):
            for j in range(BB):
                pltpu.make_async_copy(
                    table.at[idxv.at[ph, j, 0]], rows.at[ph, j],
                    gsems.at[ph, j]).wait()
                pltpu.async_copy(
                    rows.at[ph, j], acc.at[idxv.at[ph, j, 1]],
                    ssems.at[ph, j], add=True)

        def wait_scatters(ph):
            for j in range(BB):
                pltpu.make_async_copy(
                    rows.at[ph, j], acc.at[idxv.at[ph, j, 1]],
                    ssems.at[ph, j]).wait()

        load_idx(0, base)
        issue_gathers(0)
        load_idx(1, base + BB)
        issue_gathers(1)

        def pair(p, _):
            for ph in range(2):
                wait_gathers_issue_scatters(ph)
                wait_scatters(ph)
                load_idx(ph, base + (2 * p + ph + 2) * BB)
                issue_gathers(ph)
            return 0

        lax.fori_loop(0, n_chunks // 2 - 1, pair, 0)
        for ph in range(2):
            wait_gathers_issue_scatters(ph)
            wait_scatters(ph)
        plsc.subcore_barrier()
        for k in range(RPT // DRB):
            r0 = s * RPT + k * DRB
            pltpu.sync_copy(acc.at[pl.ds(r0, DRB), :], zbuf)
            pltpu.sync_copy(zbuf, out_hbm.at[c, pl.ds(r0, DRB), :])

    return agg


_agg_l1 = _make_row_agg(1, 12800, 800, 160)
_agg_l2 = _make_row_agg(2, 0, 1600, 320)


def _t0_body(deg_ref, x_ref, g1_ref):
    dis = lax.rsqrt(deg_ref[0] + deg_ref[1] + 1.0)
    g1_ref[...] = x_ref[...] * dis[:, None]


def _t0(deg, x_p):
    return pl.pallas_call(
        _t0_body,
        grid=(GRID,),
        in_specs=[
            pl.BlockSpec((2, BN), lambda i: (0, i)),
            pl.BlockSpec((BN, 16), lambda i: (i, 0)),
        ],
        out_specs=pl.BlockSpec((BN, 16), lambda i: (i, 0)),
        out_shape=jax.ShapeDtypeStruct((NP, 16), jnp.float32),
    )(deg, x_p)


def _t1_body(deg_ref, s1_ref, g1_ref, w1_ref, b1_ref, w2_ref, g2_ref):
    dis = lax.rsqrt(deg_ref[0] + deg_ref[1] + 1.0)[:, None]
    y1 = (dis * (s1_ref[0] + s1_ref[1] + g1_ref[...])) @ w1_ref[...] + b1_ref[...]
    t = jax.nn.relu(y1) @ w2_ref[...]
    g2 = t * dis
    g2_ref[0] = g2[:, :16]
    g2_ref[1] = g2[:, 16:]


def _t1(deg, s1, g1, W1, b1, W2):
    return pl.pallas_call(
        _t1_body,
        grid=(GRID,),
        in_specs=[
            pl.BlockSpec((2, BN), lambda i: (0, i)),
            pl.BlockSpec((2, BN, 16), lambda i: (0, i, 0)),
            pl.BlockSpec((BN, 16), lambda i: (i, 0)),
            pl.BlockSpec((16, 48), lambda i: (0, 0)),
            pl.BlockSpec((48,), lambda i: (0,)),
            pl.BlockSpec((48, 32), lambda i: (0, 0)),
        ],
        out_specs=pl.BlockSpec((2, BN, 16), lambda i: (0, i, 0)),
        out_shape=jax.ShapeDtypeStruct((2, NP, 16), jnp.float32),
    )(deg, s1, g1, W1, b1, W2)


def _t2_body(deg_ref, s2_ref, g2_ref, b2_ref, w3_ref, b3_ref, o_ref):
    dis = lax.rsqrt(deg_ref[0] + deg_ref[1] + 1.0)[:, None]
    ya = dis * (s2_ref[0] + g2_ref[0])
    yb = dis * (s2_ref[1] + g2_ref[1])
    h2 = jax.nn.relu(jnp.concatenate([ya, yb], axis=1) + b2_ref[...])
    o_ref[...] = h2 @ w3_ref[...] + b3_ref[...]


def _t2(deg, s2, g2, b2, W3, b3):
    return pl.pallas_call(
        _t2_body,
        grid=(GRID,),
        in_specs=[
            pl.BlockSpec((2, BN), lambda i: (0, i)),
            pl.BlockSpec((2, BN, 16), lambda i: (0, i, 0)),
            pl.BlockSpec((2, BN, 16), lambda i: (0, i, 0)),
            pl.BlockSpec((32,), lambda i: (0,)),
            pl.BlockSpec((32, 10), lambda i: (0, 0)),
            pl.BlockSpec((10,), lambda i: (0,)),
        ],
        out_specs=pl.BlockSpec((BN, 10), lambda i: (i, 0)),
        out_shape=jax.ShapeDtypeStruct((NP, 10), jnp.float32),
    )(deg, s2, g2, b2, W3, b3)


def kernel(x, edge_index, W1, b1, W2, b2, W3, b3):
    x = x.astype(jnp.float32)
    ei = edge_index.astype(jnp.int32)
    pad = jnp.full((EPAD - E,), N, jnp.int32)
    srcr = jnp.concatenate([ei[0], pad]).reshape(ER, 128)
    dstr = jnp.concatenate([ei[1], pad]).reshape(ER, 128)
    x_p = jnp.zeros((NP, 16), jnp.float32).at[:N].set(x)

    ones128 = jnp.ones((128,), jnp.float32)
    zeros1 = jnp.zeros((RPT,), jnp.float32)
    zeros16 = jnp.zeros((DRB, 16), jnp.float32)

    eidx = jnp.stack([srcr, dstr], axis=1)
    deg = _deg_sc(dstr, ones128, zeros1)
    g1 = _t0(deg, x_p)
    s1 = _agg_l1(g1.reshape(1, NP, 16), eidx, zeros16)
    g2 = _t1(deg, s1, g1, W1, b1, W2)
    s2 = _agg_l2(g2, eidx, zeros16)
    out = _t2(deg, s2, g2, b2, W3, b3)
    return out[:N]


# speedup vs baseline: 1.2684x; 1.0099x over previous
import functools

import jax
import jax.numpy as jnp
from jax import lax
from jax.experimental import pallas as pl
from jax.experimental.pallas import tpu as pltpu
from jax.experimental.pallas import tpu_sc as plsc

N = 100000
NP = 100096
E = 3200000
EPAD = 3276800
ER = EPAD // 128
RPT = NP // 16
DRB = 184
BB = 5
BN = 4352
GRID = NP // BN

_mesh = plsc.VectorSubcoreMesh(core_axis_name="c", subcore_axis_name="s")
_sc_params = pltpu.CompilerParams(use_tc_tiling_on_sc=False)


@functools.partial(
    pl.kernel,
    out_type=jax.ShapeDtypeStruct((2, NP), jnp.float32),
    mesh=_mesh,
    compiler_params=_sc_params,
    scratch_types=[
        pltpu.VMEM((128,), jnp.float32),
        pltpu.VMEM((RPT,), jnp.float32),
        pltpu.VMEM((16, 128), jnp.int32),
        pltpu.SemaphoreType.DMA((16,)),
        pltpu.VMEM_SHARED((NP,), jnp.float32),
    ],
)
def _deg_sc(dst_hbm, ones_hbm, zeros_hbm, out_hbm, ones_v, zbuf, dstv, sems, acc):
    c = lax.axis_index("c")
    s = lax.axis_index("s")
    pltpu.sync_copy(ones_hbm, ones_v)
    pltpu.sync_copy(zeros_hbm, zbuf)
    pltpu.sync_copy(zbuf, acc.at[pl.ds(s * RPT, RPT)])
    plsc.subcore_barrier()
    base = (s * 2 + c) * 800

    def chunk(i, _):
        off = base + i * 16
        pltpu.sync_copy(dst_hbm.at[pl.ds(off, 16), :], dstv)
        sd = [
            pltpu.async_copy(ones_v, acc.at[dstv.at[j]], sems.at[j], add=True)
            for j in range(16)
        ]
        for j in range(16):
            sd[j].wait()
        return 0

    lax.fori_loop(0, 50, chunk, 0)
    plsc.subcore_barrier()
    pltpu.sync_copy(acc.at[pl.ds(s * RPT, RPT)], zbuf)
    pltpu.sync_copy(zbuf, out_hbm.at[c, pl.ds(s * RPT, RPT)])


def _make_row_agg(n_tables, c_stride, s_stride, n_chunks):
    @functools.partial(
        pl.kernel,
        out_type=jax.ShapeDtypeStruct((2, NP, 16), jnp.float32),
        mesh=_mesh,
        compiler_params=_sc_params,
        scratch_types=[
            pltpu.VMEM((2, DRB, 16), jnp.float32),
            pltpu.VMEM((2, BB, 2, 128), jnp.int32),
            pltpu.VMEM((2, BB, 128, 16), jnp.float32),
            pltpu.SemaphoreType.DMA((2, BB)),
            pltpu.SemaphoreType.DMA((2, BB)),
            pltpu.SemaphoreType.DMA((2,)),
            pltpu.VMEM_SHARED((NP, 16), jnp.float32),
        ],
    )
    def agg(table_hbm, eidx_hbm, zeros_hbm, out_hbm,
            zbuf, idxv, rows, gsems, ssems, dsems, acc):
        c = lax.axis_index("c")
        s = lax.axis_index("s")
        table = table_hbm.at[c * (n_tables - 1)]
        pltpu.sync_copy(zeros_hbm, zbuf.at[0])
        pltpu.sync_copy(zeros_hbm, zbuf.at[1])
        nzc = RPT // DRB
        for k in range(nzc):
            b = k % 2
            if k >= 2:
                pltpu.make_async_copy(
                    zbuf.at[b], acc.at[pl.ds(s * RPT + (k - 2) * DRB, DRB), :],
                    dsems.at[b]).wait()
            pltpu.async_copy(
                zbuf.at[b], acc.at[pl.ds(s * RPT + k * DRB, DRB), :],
                dsems.at[b])
        for k in (nzc - 2, nzc - 1):
            pltpu.make_async_copy(
                zbuf.at[k % 2], acc.at[pl.ds(s * RPT + k * DRB, DRB), :],
                dsems.at[k % 2]).wait()
        plsc.subcore_barrier()
        base = c * c_stride + s * s_stride

        def load_idx(ph, off):
            pltpu.sync_copy(eidx_hbm.at[pl.ds(off, BB), :, :], idxv.at[ph])

        def issue_gathers(ph):
            for j in range(BB):
                pltpu.async_copy(
                    table.at[idxv.at[ph, j, 0]], rows.at[ph, j],
                    gsems.at[ph, j])

        def wait_gathers_issue_scatters(ph):
            for j in range(BB):
                pltpu.make_async_copy(
                    table.at[idxv.at[ph, j, 0]], rows.at[ph, j],
                    gsems.at[ph, j]).wait()
                pltpu.async_copy(
                    rows.at[ph, j], acc.at[idxv.at[ph, j, 1]],
                    ssems.at[ph, j], add=True)

        def wait_scatters(ph):
            for j in range(BB):
                pltpu.make_async_copy(
                    rows.at[ph, j], acc.at[idxv.at[ph, j, 1]],
                    ssems.at[ph, j]).wait()

        load_idx(0, base)
        issue_gathers(0)
        load_idx(1, base + BB)
        issue_gathers(1)

        def pair(p, _):
            for ph in range(2):
                wait_gathers_issue_scatters(ph)
                wait_scatters(ph)
                load_idx(ph, base + (2 * p + ph + 2) * BB)
                issue_gathers(ph)
            return 0

        lax.fori_loop(0, n_chunks // 2 - 1, pair, 0)
        for ph in range(2):
            wait_gathers_issue_scatters(ph)
            wait_scatters(ph)
        plsc.subcore_barrier()
        ndc = RPT // DRB
        for k in range(ndc):
            b = k % 2
            r0p = s * RPT + (k - 2) * DRB
            if k >= 2:
                pltpu.make_async_copy(
                    zbuf.at[b], out_hbm.at[c, pl.ds(r0p, DRB), :],
                    dsems.at[b]).wait()
            r0 = s * RPT + k * DRB
            pltpu.sync_copy(acc.at[pl.ds(r0, DRB), :], zbuf.at[b])
            pltpu.async_copy(
                zbuf.at[b], out_hbm.at[c, pl.ds(r0, DRB), :], dsems.at[b])
        for k in (ndc - 2, ndc - 1):
            pltpu.make_async_copy(
                zbuf.at[k % 2], out_hbm.at[c, pl.ds(k * DRB + s * RPT, DRB), :],
                dsems.at[k % 2]).wait()

    return agg


_agg_l1 = _make_row_agg(1, 12800, 800, 160)
_agg_l2 = _make_row_agg(2, 0, 1600, 320)


def _t0_body(deg_ref, x_ref, g1_ref):
    dis = lax.rsqrt(deg_ref[0] + deg_ref[1] + 1.0)
    g1_ref[...] = x_ref[...] * dis[:, None]


def _t0(deg, x_p):
    return pl.pallas_call(
        _t0_body,
        grid=(GRID,),
        in_specs=[
            pl.BlockSpec((2, BN), lambda i: (0, i)),
            pl.BlockSpec((BN, 16), lambda i: (i, 0)),
        ],
        out_specs=pl.BlockSpec((BN, 16), lambda i: (i, 0)),
        out_shape=jax.ShapeDtypeStruct((NP, 16), jnp.float32),
    )(deg, x_p)


def _t1_body(deg_ref, s1_ref, g1_ref, w1_ref, b1_ref, w2_ref, g2_ref):
    dis = lax.rsqrt(deg_ref[0] + deg_ref[1] + 1.0)[:, None]
    y1 = (dis * (s1_ref[0] + s1_ref[1] + g1_ref[...])) @ w1_ref[...] + b1_ref[...]
    t = jax.nn.relu(y1) @ w2_ref[...]
    g2 = t * dis
    g2_ref[0] = g2[:, :16]
    g2_ref[1] = g2[:, 16:]


def _t1(deg, s1, g1, W1, b1, W2):
    return pl.pallas_call(
        _t1_body,
        grid=(GRID,),
        in_specs=[
            pl.BlockSpec((2, BN), lambda i: (0, i)),
            pl.BlockSpec((2, BN, 16), lambda i: (0, i, 0)),
            pl.BlockSpec((BN, 16), lambda i: (i, 0)),
            pl.BlockSpec((16, 48), lambda i: (0, 0)),
            pl.BlockSpec((48,), lambda i: (0,)),
            pl.BlockSpec((48, 32), lambda i: (0, 0)),
        ],
        out_specs=pl.BlockSpec((2, BN, 16), lambda i: (0, i, 0)),
        out_shape=jax.ShapeDtypeStruct((2, NP, 16), jnp.float32),
    )(deg, s1, g1, W1, b1, W2)


def _t2_body(deg_ref, s2_ref, g2_ref, b2_ref, w3_ref, b3_ref, o_ref):
    dis = lax.rsqrt(deg_ref[0] + deg_ref[1] + 1.0)[:, None]
    ya = dis * (s2_ref[0] + g2_ref[0])
    yb = dis * (s2_ref[1] + g2_ref[1])
    h2 = jax.nn.relu(jnp.concatenate([ya, yb], axis=1) + b2_ref[...])
    o_ref[...] = h2 @ w3_ref[...] + b3_ref[...]


def _t2(deg, s2, g2, b2, W3, b3):
    return pl.pallas_call(
        _t2_body,
        grid=(GRID,),
        in_specs=[
            pl.BlockSpec((2, BN), lambda i: (0, i)),
            pl.BlockSpec((2, BN, 16), lambda i: (0, i, 0)),
            pl.BlockSpec((2, BN, 16), lambda i: (0, i, 0)),
            pl.BlockSpec((32,), lambda i: (0,)),
            pl.BlockSpec((32, 10), lambda i: (0, 0)),
            pl.BlockSpec((10,), lambda i: (0,)),
        ],
        out_specs=pl.BlockSpec((BN, 10), lambda i: (i, 0)),
        out_shape=jax.ShapeDtypeStruct((NP, 10), jnp.float32),
    )(deg, s2, g2, b2, W3, b3)


def kernel(x, edge_index, W1, b1, W2, b2, W3, b3):
    x = x.astype(jnp.float32)
    ei = edge_index.astype(jnp.int32)
    pad = jnp.full((EPAD - E,), N, jnp.int32)
    srcr = jnp.concatenate([ei[0], pad]).reshape(ER, 128)
    dstr = jnp.concatenate([ei[1], pad]).reshape(ER, 128)
    x_p = jnp.zeros((NP, 16), jnp.float32).at[:N].set(x)

    ones128 = jnp.ones((128,), jnp.float32)
    zeros1 = jnp.zeros((RPT,), jnp.float32)
    zeros16 = jnp.zeros((DRB, 16), jnp.float32)

    eidx = jnp.stack([srcr, dstr], axis=1)
    deg = _deg_sc(dstr, ones128, zeros1)
    g1 = _t0(deg, x_p)
    s1 = _agg_l1(g1.reshape(1, NP, 16), eidx, zeros16)
    g2 = _t1(deg, s1, g1, W1, b1, W2)
    s2 = _agg_l2(g2, eidx, zeros16)
    out = _t2(deg, s2, g2, b2, W3, b3)
    return out[:N]
